# Initial kernel scaffold; baseline (speedup 1.0000x reference)
import jax
import jax.numpy as jnp
from jax.experimental import pallas as pl


def kernel(x_table, x_column, ei_contains, ei_foreign_key, ei_rev_contains, enc_t_W, enc_t_b, enc_c_W, enc_c_b, Wl, bl, Wr, cls_t_W, cls_t_b, cls_c_W, cls_c_b):
    raise NotImplementedError("write your pallas kernel here")


# speedup vs baseline: 4.4769x; 4.4769x over previous
import functools

import jax
import jax.numpy as jnp
from jax import lax
from jax.experimental import pallas as pl
from jax.experimental.pallas import tpu as pltpu
from jax.experimental.pallas import tpu_sc as plsc

N = 50000
H = 64
L = 3

BLK = 512
NP = 50176
HALF = NP // 2
ACC_ROWS = 25600
TRASH = HALF
EC = 128
NTILES = 32
ZROWS = ACC_ROWS // 16

E_FK_PAD = 802816
E_C_PAD = 53248


def _dst_localize(d, c):
    lo = c * HALF
    local = d - lo
    ok = (local >= 0) & (local < HALF)
    trash = TRASH + (d & 511)
    return jnp.where(ok, local, trash)


@functools.lru_cache(maxsize=None)
def _make_segsum(e_pad):
    n_chunks = e_pad // (NTILES * EC)
    mesh = plsc.VectorSubcoreMesh(core_axis_name="c", subcore_axis_name="s")

    @functools.partial(
        pl.kernel,
        mesh=mesh,
        compiler_params=pltpu.CompilerParams(use_tc_tiling_on_sc=False),
        out_type=jax.ShapeDtypeStruct((2, HALF, H), jnp.float32),
        scratch_types=[
            pltpu.VMEM_SHARED((ACC_ROWS, H), jnp.float32),
            pltpu.VMEM((EC,), jnp.int32),
            pltpu.VMEM((EC,), jnp.int32),
            pltpu.VMEM((EC,), jnp.int32),
            pltpu.VMEM((EC, H), jnp.float32),
            pltpu.SemaphoreType.DMA,
        ],
    )
    def seg(src_hbm, dst_hbm, x_hbm, out_hbm, acc, sidx, draw, dloc, rows, sem):
        c = lax.axis_index("c")
        s = lax.axis_index("s")

        def zrow(i, carry):
            for h4 in range(H // 16):
                rows[i, pl.ds(h4 * 16, 16)] = jnp.zeros((16,), jnp.float32)
            return carry
        lax.fori_loop(0, EC, zrow, 0)
        for j in range(ZROWS // 100):
            pltpu.sync_copy(rows.at[pl.ds(0, 100), :],
                            acc.at[pl.ds(s * ZROWS + j * 100, 100), :])
        plsc.subcore_barrier()

        tbase = (s * 2 + c) * (e_pad // NTILES)

        def body(i, carry):
            base = tbase + i * EC
            pltpu.sync_copy(src_hbm.at[pl.ds(base, EC)], sidx)
            pltpu.sync_copy(dst_hbm.at[pl.ds(base, EC)], draw)
            for k in range(EC // 16):
                d = draw[pl.ds(k * 16, 16)]
                dloc[pl.ds(k * 16, 16)] = _dst_localize(d, c)
            pltpu.async_copy(x_hbm.at[sidx], rows, sem).wait()
            pltpu.sync_copy(rows, acc.at[dloc], add=True)
            return carry
        lax.fori_loop(0, n_chunks, body, 0)
        plsc.subcore_barrier()

        for j in range(12):
            r0 = s * (HALF // 16) + j * EC
            pltpu.sync_copy(acc.at[pl.ds(r0, EC), :], rows)
            pltpu.sync_copy(rows, out_hbm.at[c, pl.ds(r0, EC), :])
        r0 = s * (HALF // 16) + 12 * EC
        pltpu.sync_copy(acc.at[pl.ds(r0, 32), :], rows.at[pl.ds(0, 32), :])
        pltpu.sync_copy(rows.at[pl.ds(0, 32), :], out_hbm.at[c, pl.ds(r0, 32), :])

    return seg


@functools.lru_cache(maxsize=None)
def _make_segcnt(e_pad):
    n_chunks = e_pad // (NTILES * EC)
    mesh = plsc.VectorSubcoreMesh(core_axis_name="c", subcore_axis_name="s")

    @functools.partial(
        pl.kernel,
        mesh=mesh,
        compiler_params=pltpu.CompilerParams(use_tc_tiling_on_sc=False),
        out_type=jax.ShapeDtypeStruct((NP,), jnp.float32),
        scratch_types=[
            pltpu.VMEM_SHARED((ACC_ROWS,), jnp.float32),
            pltpu.VMEM((EC,), jnp.int32),
            pltpu.VMEM((EC,), jnp.int32),
            pltpu.VMEM((EC,), jnp.float32),
            pltpu.VMEM((ZROWS,), jnp.float32),
        ],
    )
    def cnt(dst_hbm, out_hbm, acc, draw, dloc, ones, wbuf):
        c = lax.axis_index("c")
        s = lax.axis_index("s")

        for k in range(EC // 16):
            ones[pl.ds(k * 16, 16)] = jnp.ones((16,), jnp.float32)

        def zb(i, carry):
            wbuf[pl.ds(i * 16, 16)] = jnp.zeros((16,), jnp.float32)
            return carry
        lax.fori_loop(0, ZROWS // 16, zb, 0)
        pltpu.sync_copy(wbuf, acc.at[pl.ds(s * ZROWS, ZROWS)])
        plsc.subcore_barrier()

        tbase = (s * 2 + c) * (e_pad // NTILES)

        def body(i, carry):
            base = tbase + i * EC
            pltpu.sync_copy(dst_hbm.at[pl.ds(base, EC)], draw)
            for k in range(EC // 16):
                d = draw[pl.ds(k * 16, 16)]
                dloc[pl.ds(k * 16, 16)] = _dst_localize(d, c)
            pltpu.sync_copy(ones, acc.at[dloc], add=True)
            return carry
        lax.fori_loop(0, n_chunks, body, 0)
        plsc.subcore_barrier()

        r0 = s * (HALF // 16)
        pltpu.sync_copy(acc.at[pl.ds(r0, HALF // 16)], wbuf.at[pl.ds(0, HALF // 16)])
        pltpu.sync_copy(wbuf.at[pl.ds(0, HALF // 16)],
                        out_hbm.at[pl.ds(c * HALF + r0, HALF // 16)])

    return cnt


def _enc_body(x_ref, w_ref, b_ref, o_ref):
    o_ref[...] = x_ref[...] * w_ref[...] + b_ref[...]


def _encode(x, w, b):
    grid = (NP // BLK,)
    return pl.pallas_call(
        _enc_body,
        grid=grid,
        in_specs=[
            pl.BlockSpec((BLK, 1), lambda i: (i, 0)),
            pl.BlockSpec((1, H), lambda i: (0, 0)),
            pl.BlockSpec((1, H), lambda i: (0, 0)),
        ],
        out_specs=pl.BlockSpec((BLK, H), lambda i: (i, 0)),
        out_shape=jax.ShapeDtypeStruct((NP, H), jnp.float32),
    )(x, w, b)


def _layer_body(ht, hc, sc_, sf, sr, cc, cf, cr, wl, bl_, wr, oht, ohc):
    mc = sc_[...] * (1.0 / jnp.maximum(cc[...], 1.0))
    mf = sf[...] * (1.0 / jnp.maximum(cf[...], 1.0))
    mr = sr[...] * (1.0 / jnp.maximum(cr[...], 1.0))
    wsum = wr[0] + wr[1]
    col = (jnp.dot(mc, wl[0], preferred_element_type=jnp.float32)
           + jnp.dot(mf, wl[1], preferred_element_type=jnp.float32)
           + jnp.dot(hc[...], wsum, preferred_element_type=jnp.float32)
           + bl_[0] + bl_[1]) * 0.5
    tab = (jnp.dot(mr, wl[2], preferred_element_type=jnp.float32)
           + bl_[2]
           + jnp.dot(ht[...], wr[2], preferred_element_type=jnp.float32))
    oht[...] = jnp.maximum(ht[...] + col, 0.0)
    ohc[...] = jnp.maximum(hc[...] + tab, 0.0)


def _layer(ht, hc, sc_, sf, sr, cc, cf, cr, wl, bl_, wr):
    grid = (NP // BLK,)
    mat = pl.BlockSpec((BLK, H), lambda i: (i, 0))
    vec = pl.BlockSpec((BLK, 1), lambda i: (i, 0))
    return pl.pallas_call(
        _layer_body,
        grid=grid,
        in_specs=[mat, mat, mat, mat, mat, vec, vec, vec,
                  pl.BlockSpec((3, H, H), lambda i: (0, 0, 0)),
                  pl.BlockSpec((3, 1, H), lambda i: (0, 0, 0)),
                  pl.BlockSpec((3, H, H), lambda i: (0, 0, 0))],
        out_specs=[mat, mat],
        out_shape=[jax.ShapeDtypeStruct((NP, H), jnp.float32),
                   jax.ShapeDtypeStruct((NP, H), jnp.float32)],
    )(ht, hc, sc_, sf, sr, cc, cf, cr, wl, bl_, wr)


def _cls_body(ht, hc, wt, bt, wc, bc, ot, oc):
    ot[...] = jax.nn.sigmoid(
        jnp.dot(ht[...], wt[...], preferred_element_type=jnp.float32) + bt[...])
    oc[...] = jax.nn.sigmoid(
        jnp.dot(hc[...], wc[...], preferred_element_type=jnp.float32) + bc[...])


def _classify(ht, hc, wt, bt, wc, bc):
    grid = (NP // BLK,)
    mat = pl.BlockSpec((BLK, H), lambda i: (i, 0))
    return pl.pallas_call(
        _cls_body,
        grid=grid,
        in_specs=[mat, mat,
                  pl.BlockSpec((H, 1), lambda i: (0, 0)),
                  pl.BlockSpec((1, 1), lambda i: (0, 0)),
                  pl.BlockSpec((H, 1), lambda i: (0, 0)),
                  pl.BlockSpec((1, 1), lambda i: (0, 0))],
        out_specs=[pl.BlockSpec((BLK, 1), lambda i: (i, 0)),
                   pl.BlockSpec((BLK, 1), lambda i: (i, 0))],
        out_shape=[jax.ShapeDtypeStruct((NP, 1), jnp.float32),
                   jax.ShapeDtypeStruct((NP, 1), jnp.float32)],
    )(ht, hc, wt, bt, wc, bc)


def _pad_edges(ei, e_pad):
    e = ei.shape[1]
    pad = e_pad - e
    src = jnp.concatenate([ei[0], jnp.zeros((pad,), jnp.int32)])
    dst = jnp.concatenate([ei[1], jnp.full((pad,), -1, jnp.int32)])
    return src, dst


def kernel(x_table, x_column, ei_contains, ei_foreign_key, ei_rev_contains,
           enc_t_W, enc_t_b, enc_c_W, enc_c_b, Wl, bl, Wr,
           cls_t_W, cls_t_b, cls_c_W, cls_c_b):
    xt = jnp.pad(x_table.astype(jnp.float32), (0, NP - N)).reshape(NP, 1)
    xc = jnp.pad(x_column.astype(jnp.float32), (0, NP - N)).reshape(NP, 1)
    h_t = _encode(xt, enc_t_W, enc_t_b.reshape(1, H))
    h_c = _encode(xc, enc_c_W, enc_c_b.reshape(1, H))

    src_c, dst_c = _pad_edges(ei_contains, E_C_PAD)
    src_f, dst_f = _pad_edges(ei_foreign_key, E_FK_PAD)
    src_r, dst_r = _pad_edges(ei_rev_contains, E_C_PAD)

    segc = _make_segsum(E_C_PAD)
    segf = _make_segsum(E_FK_PAD)
    cntc = _make_segcnt(E_C_PAD)
    cntf = _make_segcnt(E_FK_PAD)

    cc = cntc(dst_c).reshape(NP, 1)
    cf = cntf(dst_f).reshape(NP, 1)
    cr = cntc(dst_r).reshape(NP, 1)

    for l in range(L):
        s_c = segc(src_c, dst_c, h_t).reshape(NP, H)
        s_f = segf(src_f, dst_f, h_c).reshape(NP, H)
        s_r = segc(src_r, dst_r, h_c).reshape(NP, H)
        h_t, h_c = _layer(h_t, h_c, s_c, s_f, s_r, cc, cf, cr,
                          Wl[l], bl[l].reshape(3, 1, H), Wr[l])

    t_out, c_out = _classify(h_t, h_c, cls_t_W, cls_t_b.reshape(1, 1),
                             cls_c_W, cls_c_b.reshape(1, 1))
    return t_out.reshape(NP)[:N], c_out.reshape(NP)[:N]


# speedup vs baseline: 4.8697x; 1.0878x over previous
import functools

import jax
import jax.numpy as jnp
from jax import lax
from jax.experimental import pallas as pl
from jax.experimental.pallas import tpu as pltpu
from jax.experimental.pallas import tpu_sc as plsc

N = 50000
H = 64
L = 3

BLK = 512
NP = 50176
HALF = NP // 2
ACC_ROWS = 25600
TRASH = HALF
EC = 128
NTILES = 32
ZROWS = ACC_ROWS // 16

E_FK_PAD = 802816
E_C_PAD = 57344


def _dst_localize(d, c):
    lo = c * HALF
    local = d - lo
    ok = (local >= 0) & (local < HALF)
    trash = TRASH + (d & 511)
    return jnp.where(ok, local, trash)


@functools.lru_cache(maxsize=None)
def _make_segsum(e_pad):
    n_chunks = e_pad // (NTILES * EC)
    mesh = plsc.VectorSubcoreMesh(core_axis_name="c", subcore_axis_name="s")

    @functools.partial(
        pl.kernel,
        mesh=mesh,
        compiler_params=pltpu.CompilerParams(use_tc_tiling_on_sc=False),
        out_type=jax.ShapeDtypeStruct((2, HALF, H), jnp.float32),
        scratch_types=[
            pltpu.VMEM_SHARED((ACC_ROWS, H), jnp.float32),
            pltpu.VMEM((2, EC), jnp.int32),
            pltpu.VMEM((2, EC), jnp.int32),
            pltpu.VMEM((2, EC), jnp.int32),
            pltpu.VMEM((2, EC, H), jnp.float32),
            pltpu.SemaphoreType.DMA((2,)),
            pltpu.SemaphoreType.DMA((2,)),
            pltpu.SemaphoreType.DMA((2,)),
        ],
    )
    def seg(src_hbm, dst_hbm, x_hbm, out_hbm, acc,
            sidx, draw, dloc, rows, isem, gsem, ssem):
        c = lax.axis_index("c")
        s = lax.axis_index("s")

        def zrow(i, carry):
            for b in range(2):
                for h4 in range(H // 16):
                    rows[b, i, pl.ds(h4 * 16, 16)] = jnp.zeros((16,), jnp.float32)
            return carry
        lax.fori_loop(0, EC, zrow, 0)
        for j in range(ZROWS // 100):
            pltpu.sync_copy(rows.at[0, pl.ds(0, 100), :],
                            acc.at[pl.ds(s * ZROWS + j * 100, 100), :])
        plsc.subcore_barrier()

        tbase = (s * 2 + c) * (e_pad // NTILES)

        def idx_start(ci, b):
            base = tbase + ci * EC
            pltpu.async_copy(src_hbm.at[pl.ds(base, EC)], sidx.at[b], isem.at[b])
            pltpu.async_copy(dst_hbm.at[pl.ds(base, EC)], draw.at[b], isem.at[b])

        def idx_wait(b):
            pltpu.make_async_copy(src_hbm.at[pl.ds(0, EC)], sidx.at[b],
                                  isem.at[b]).wait()
            pltpu.make_async_copy(dst_hbm.at[pl.ds(0, EC)], draw.at[b],
                                  isem.at[b]).wait()

        def gather_start(b):
            pltpu.async_copy(x_hbm.at[sidx.at[b]], rows.at[b], gsem.at[b])

        def gather_wait(b):
            pltpu.make_async_copy(x_hbm.at[sidx.at[b]], rows.at[b],
                                  gsem.at[b]).wait()

        def scat_start(b):
            pltpu.async_copy(rows.at[b], acc.at[dloc.at[b]], ssem.at[b],
                             add=True)

        def scat_wait(b):
            pltpu.make_async_copy(rows.at[b], acc.at[dloc.at[b]],
                                  ssem.at[b]).wait()

        def dloc_compute(b):
            for k in range(EC // 16):
                d = draw[b, pl.ds(k * 16, 16)]
                dloc[b, pl.ds(k * 16, 16)] = _dst_localize(d, c)

        for k in range(EC // 16):
            t16 = TRASH + jnp.arange(16, dtype=jnp.int32) + k * 16
            dloc[0, pl.ds(k * 16, 16)] = t16
            dloc[1, pl.ds(k * 16, 16)] = t16
        idx_start(0, 0)
        idx_start(1, 1)
        scat_start(0)
        scat_start(1)
        idx_wait(0)
        scat_wait(0)
        dloc_compute(0)
        gather_start(0)

        def body(g, carry):
            c0 = 2 * g
            for b in range(2):
                gather_wait(b)
                scat_start(b)
                idx_start(c0 + b + 2, b)
                b1 = 1 - b
                idx_wait(b1)
                scat_wait(b1)
                dloc_compute(b1)
                gather_start(b1)
            return carry
        lax.fori_loop(0, n_chunks // 2 - 1, body, 0)

        gather_wait(0)
        scat_start(0)
        idx_wait(1)
        scat_wait(1)
        dloc_compute(1)
        gather_start(1)
        gather_wait(1)
        scat_start(1)
        scat_wait(0)
        scat_wait(1)
        plsc.subcore_barrier()

        for j in range(12):
            r0 = s * (HALF // 16) + j * EC
            pltpu.sync_copy(acc.at[pl.ds(r0, EC), :], rows.at[0])
            pltpu.sync_copy(rows.at[0], out_hbm.at[c, pl.ds(r0, EC), :])
        r0 = s * (HALF // 16) + 12 * EC
        pltpu.sync_copy(acc.at[pl.ds(r0, 32), :], rows.at[0, pl.ds(0, 32), :])
        pltpu.sync_copy(rows.at[0, pl.ds(0, 32), :],
                        out_hbm.at[c, pl.ds(r0, 32), :])

    return seg


@functools.lru_cache(maxsize=None)
def _make_segcnt(e_pad):
    n_chunks = e_pad // (NTILES * EC)
    mesh = plsc.VectorSubcoreMesh(core_axis_name="c", subcore_axis_name="s")

    @functools.partial(
        pl.kernel,
        mesh=mesh,
        compiler_params=pltpu.CompilerParams(use_tc_tiling_on_sc=False),
        out_type=jax.ShapeDtypeStruct((NP,), jnp.float32),
        scratch_types=[
            pltpu.VMEM_SHARED((ACC_ROWS,), jnp.float32),
            pltpu.VMEM((EC,), jnp.int32),
            pltpu.VMEM((EC,), jnp.int32),
            pltpu.VMEM((EC,), jnp.float32),
            pltpu.VMEM((ZROWS,), jnp.float32),
        ],
    )
    def cnt(dst_hbm, out_hbm, acc, draw, dloc, ones, wbuf):
        c = lax.axis_index("c")
        s = lax.axis_index("s")

        for k in range(EC // 16):
            ones[pl.ds(k * 16, 16)] = jnp.ones((16,), jnp.float32)

        def zb(i, carry):
            wbuf[pl.ds(i * 16, 16)] = jnp.zeros((16,), jnp.float32)
            return carry
        lax.fori_loop(0, ZROWS // 16, zb, 0)
        pltpu.sync_copy(wbuf, acc.at[pl.ds(s * ZROWS, ZROWS)])
        plsc.subcore_barrier()

        tbase = (s * 2 + c) * (e_pad // NTILES)

        def body(i, carry):
            base = tbase + i * EC
            pltpu.sync_copy(dst_hbm.at[pl.ds(base, EC)], draw)
            for k in range(EC // 16):
                d = draw[pl.ds(k * 16, 16)]
                dloc[pl.ds(k * 16, 16)] = _dst_localize(d, c)
            pltpu.sync_copy(ones, acc.at[dloc], add=True)
            return carry
        lax.fori_loop(0, n_chunks, body, 0)
        plsc.subcore_barrier()

        r0 = s * (HALF // 16)
        pltpu.sync_copy(acc.at[pl.ds(r0, HALF // 16)], wbuf.at[pl.ds(0, HALF // 16)])
        pltpu.sync_copy(wbuf.at[pl.ds(0, HALF // 16)],
                        out_hbm.at[pl.ds(c * HALF + r0, HALF // 16)])

    return cnt


def _enc_body(x_ref, w_ref, b_ref, o_ref):
    o_ref[...] = x_ref[...] * w_ref[...] + b_ref[...]


def _encode(x, w, b):
    grid = (NP // BLK,)
    return pl.pallas_call(
        _enc_body,
        grid=grid,
        in_specs=[
            pl.BlockSpec((BLK, 1), lambda i: (i, 0)),
            pl.BlockSpec((1, H), lambda i: (0, 0)),
            pl.BlockSpec((1, H), lambda i: (0, 0)),
        ],
        out_specs=pl.BlockSpec((BLK, H), lambda i: (i, 0)),
        out_shape=jax.ShapeDtypeStruct((NP, H), jnp.float32),
    )(x, w, b)


def _layer_body(ht, hc, sc_, sf, sr, cc, cf, cr, wl, bl_, wr, oht, ohc):
    mc = sc_[...] * (1.0 / jnp.maximum(cc[...], 1.0))
    mf = sf[...] * (1.0 / jnp.maximum(cf[...], 1.0))
    mr = sr[...] * (1.0 / jnp.maximum(cr[...], 1.0))
    wsum = wr[0] + wr[1]
    col = (jnp.dot(mc, wl[0], preferred_element_type=jnp.float32)
           + jnp.dot(mf, wl[1], preferred_element_type=jnp.float32)
           + jnp.dot(hc[...], wsum, preferred_element_type=jnp.float32)
           + bl_[0] + bl_[1]) * 0.5
    tab = (jnp.dot(mr, wl[2], preferred_element_type=jnp.float32)
           + bl_[2]
           + jnp.dot(ht[...], wr[2], preferred_element_type=jnp.float32))
    oht[...] = jnp.maximum(ht[...] + col, 0.0)
    ohc[...] = jnp.maximum(hc[...] + tab, 0.0)


def _layer(ht, hc, sc_, sf, sr, cc, cf, cr, wl, bl_, wr):
    grid = (NP // BLK,)
    mat = pl.BlockSpec((BLK, H), lambda i: (i, 0))
    vec = pl.BlockSpec((BLK, 1), lambda i: (i, 0))
    return pl.pallas_call(
        _layer_body,
        grid=grid,
        in_specs=[mat, mat, mat, mat, mat, vec, vec, vec,
                  pl.BlockSpec((3, H, H), lambda i: (0, 0, 0)),
                  pl.BlockSpec((3, 1, H), lambda i: (0, 0, 0)),
                  pl.BlockSpec((3, H, H), lambda i: (0, 0, 0))],
        out_specs=[mat, mat],
        out_shape=[jax.ShapeDtypeStruct((NP, H), jnp.float32),
                   jax.ShapeDtypeStruct((NP, H), jnp.float32)],
    )(ht, hc, sc_, sf, sr, cc, cf, cr, wl, bl_, wr)


def _cls_body(ht, hc, wt, bt, wc, bc, ot, oc):
    ot[...] = jax.nn.sigmoid(
        jnp.dot(ht[...], wt[...], preferred_element_type=jnp.float32) + bt[...])
    oc[...] = jax.nn.sigmoid(
        jnp.dot(hc[...], wc[...], preferred_element_type=jnp.float32) + bc[...])


def _classify(ht, hc, wt, bt, wc, bc):
    grid = (NP // BLK,)
    mat = pl.BlockSpec((BLK, H), lambda i: (i, 0))
    return pl.pallas_call(
        _cls_body,
        grid=grid,
        in_specs=[mat, mat,
                  pl.BlockSpec((H, 1), lambda i: (0, 0)),
                  pl.BlockSpec((1, 1), lambda i: (0, 0)),
                  pl.BlockSpec((H, 1), lambda i: (0, 0)),
                  pl.BlockSpec((1, 1), lambda i: (0, 0))],
        out_specs=[pl.BlockSpec((BLK, 1), lambda i: (i, 0)),
                   pl.BlockSpec((BLK, 1), lambda i: (i, 0))],
        out_shape=[jax.ShapeDtypeStruct((NP, 1), jnp.float32),
                   jax.ShapeDtypeStruct((NP, 1), jnp.float32)],
    )(ht, hc, wt, bt, wc, bc)


def _pad_edges(ei, e_pad):
    e = ei.shape[1]
    pad = e_pad - e
    src = jnp.concatenate([ei[0], jnp.zeros((pad,), jnp.int32)])
    dst = jnp.concatenate([ei[1], jnp.full((pad,), -1, jnp.int32)])
    return src, dst


def kernel(x_table, x_column, ei_contains, ei_foreign_key, ei_rev_contains,
           enc_t_W, enc_t_b, enc_c_W, enc_c_b, Wl, bl, Wr,
           cls_t_W, cls_t_b, cls_c_W, cls_c_b):
    xt = jnp.pad(x_table.astype(jnp.float32), (0, NP - N)).reshape(NP, 1)
    xc = jnp.pad(x_column.astype(jnp.float32), (0, NP - N)).reshape(NP, 1)
    h_t = _encode(xt, enc_t_W, enc_t_b.reshape(1, H))
    h_c = _encode(xc, enc_c_W, enc_c_b.reshape(1, H))

    src_c, dst_c = _pad_edges(ei_contains, E_C_PAD)
    src_f, dst_f = _pad_edges(ei_foreign_key, E_FK_PAD)
    src_r, dst_r = _pad_edges(ei_rev_contains, E_C_PAD)

    segc = _make_segsum(E_C_PAD)
    segf = _make_segsum(E_FK_PAD)
    cntc = _make_segcnt(E_C_PAD)
    cntf = _make_segcnt(E_FK_PAD)

    cc = cntc(dst_c).reshape(NP, 1)
    cf = cntf(dst_f).reshape(NP, 1)
    cr = cntc(dst_r).reshape(NP, 1)

    for l in range(L):
        s_c = segc(src_c, dst_c, h_t).reshape(NP, H)
        s_f = segf(src_f, dst_f, h_c).reshape(NP, H)
        s_r = segc(src_r, dst_r, h_c).reshape(NP, H)
        h_t, h_c = _layer(h_t, h_c, s_c, s_f, s_r, cc, cf, cr,
                          Wl[l], bl[l].reshape(3, 1, H), Wr[l])

    t_out, c_out = _classify(h_t, h_c, cls_t_W, cls_t_b.reshape(1, 1),
                             cls_c_W, cls_c_b.reshape(1, 1))
    return t_out.reshape(NP)[:N], c_out.reshape(NP)[:N]
